# Initial kernel scaffold; baseline (speedup 1.0000x reference)
import jax
import jax.numpy as jnp
from jax.experimental import pallas as pl


def kernel(mem, feats, delta, idx, Wf, Wz, wd, bd, log_temp, Wg1, bg1, Wg2, bg2, W_ih, W_hh, b_ih, b_hh, ln_g, ln_b, Wd1, bd1, Wd2, bd2, Wd3, bd3):
    raise NotImplementedError("write your pallas kernel here")


# speedup vs baseline: 1.4381x; 1.4381x over previous
import functools

import jax
import jax.numpy as jnp
from jax import lax
from jax.experimental import pallas as pl
from jax.experimental.pallas import tpu as pltpu
from jax.experimental.pallas import tpu_sc as plsc

TAU = 0.3

M = 262144
N = 500000
D = 64
P = 16

NCORE = 2
NSUB = 16
NPAD = 524288
NW_SC1 = NPAD // (NCORE * NSUB)
CH_SC1 = 2048
NT = NPAD // NSUB
NB = 16
RB = M // NB
SH = 14
CHUNK = 128
CAP = NT + CHUNK
ROWW = 80
NTRASH = 64


def _sc_mesh():
    return plsc.VectorSubcoreMesh(core_axis_name="c", subcore_axis_name="s")


_SC_PARAMS = pltpu.CompilerParams(use_tc_tiling_on_sc=False,
                                  needs_layout_passes=False)


def _zg_body(zpn_hbm, idxp_hbm, out_hbm, idxc, rows, sem):
    wid = lax.axis_index("s") * NCORE + lax.axis_index("c")
    base = wid * NW_SC1
    for c in range(NW_SC1 // CH_SC1):
        pltpu.sync_copy(idxp_hbm.at[pl.ds(base + c * CH_SC1, CH_SC1)], idxc)
        pltpu.async_copy(zpn_hbm.at[idxc], rows, sem).wait()
        pltpu.sync_copy(rows, out_hbm.at[pl.ds(base + c * CH_SC1, CH_SC1)])


def _zg_gather(zpn, idxp):
    kfn = pl.kernel(
        _zg_body,
        out_type=jax.ShapeDtypeStruct((NPAD, P), jnp.float32),
        mesh=_sc_mesh(),
        scratch_types=[
            pltpu.VMEM((CH_SC1,), jnp.int32),
            pltpu.VMEM((CH_SC1, P), jnp.float32),
            pltpu.SemaphoreType.DMA,
        ],
        compiler_params=_SC_PARAMS,
    )
    return kfn(zpn, idxp)


ISEG = 16384
NSEG = NT // ISEG
RING = 512
ZROWS = 128


def _seg_body(msgext_hbm, idxp_hbm, out_hbm,
              ibuf, ringpid, ringvloc, pidchunk, vlocchunk, rowbuf, zbuf,
              msgacc, gsem, zsem):
    cid = lax.axis_index("c")
    sid = lax.axis_index("s")

    lanes = lax.iota(jnp.int32, 16)
    pidsent = lanes * 64
    vlocsent = RB + lanes * 4
    rpt = RB // NSUB

    z16 = jnp.zeros((16,), jnp.float32)

    def _zrow(r, _):
        for c5 in range(ROWW // 16):
            zbuf[r, pl.ds(c5 * 16, 16)] = z16
        return 0

    lax.fori_loop(0, ZROWS, _zrow, 0)

    def _zero_acc():
        return [pltpu.async_copy(
            zbuf, msgacc.at[pl.ds(sid * rpt + k * ZROWS, ZROWS)], zsem)
            for k in range(rpt // ZROWS)]

    def _process(proc):
        slot = proc & (RING - 1)
        for k in range(CHUNK // 16):
            pidchunk[pl.ds(k * 16, 16)] = ringpid[pl.ds(slot + k * 16, 16)]
            vlocchunk[pl.ds(k * 16, 16)] = ringvloc[pl.ds(slot + k * 16, 16)]
        pltpu.async_copy(msgext_hbm.at[pidchunk], rowbuf, gsem).wait()
        pltpu.sync_copy(rowbuf, msgacc.at[vlocchunk], add=True)

    zdescs = _zero_acc()

    for p in range(NB // NCORE):
        b = p * NCORE + cid
        lo = b * RB

        for dsc in zdescs:
            dsc.wait()
        plsc.subcore_barrier()

        woff = jnp.int32(0)
        proc = jnp.int32(0)
        for g in range(NSEG):
            pltpu.sync_copy(
                idxp_hbm.at[pl.ds(sid * NT + g * ISEG, ISEG)], ibuf)
            gbase = sid * NT + g * ISEG

            def _scan(i, carry, gbase=gbase, b=b, lo=lo):
                woff, proc = carry
                v = ibuf[pl.ds(i * 16, 16)]
                pidv = lanes + (gbase + i * 16)
                msk = (lax.shift_right_logical(v, SH) == b) & (pidv < N)
                mi = msk.astype(jnp.int32)
                tgt = (woff + plsc.cumsum(mi) - 1) & (RING - 1)
                plsc.store_scatter(ringpid, [tgt], pidv, mask=msk)
                plsc.store_scatter(ringvloc, [tgt], v - lo, mask=msk)
                woff = woff + jnp.sum(mi)

                def _do():
                    _process(proc)
                    return proc + CHUNK

                proc = lax.cond(woff - proc >= CHUNK, _do, lambda: proc)
                return woff, proc

            woff, proc = lax.fori_loop(0, ISEG // 16, _scan, (woff, proc))

        for k in range(CHUNK // 16):
            plsc.store_scatter(
                ringpid, [(woff + lanes + k * 16) & (RING - 1)], pidsent)
            plsc.store_scatter(
                ringvloc, [(woff + lanes + k * 16) & (RING - 1)], vlocsent)

        def _fin(j, pr):
            _process(pr)
            return pr + CHUNK

        nfinal = lax.shift_right_logical(woff - proc + (CHUNK - 1), 7)
        lax.fori_loop(0, nfinal, _fin, proc)
        plsc.subcore_barrier()

        pltpu.sync_copy(
            msgacc.at[pl.ds(sid * rpt, rpt)],
            out_hbm.at[pl.ds(lo + sid * rpt, rpt)])
        if p != NB // NCORE - 1:
            zdescs = _zero_acc()


def _seg_sum(msgext, idxp):
    kfn = pl.kernel(
        _seg_body,
        out_type=jax.ShapeDtypeStruct((M, ROWW), jnp.float32),
        mesh=_sc_mesh(),
        scratch_types=[
            pltpu.VMEM((ISEG,), jnp.int32),
            pltpu.VMEM((RING,), jnp.int32),
            pltpu.VMEM((RING,), jnp.int32),
            pltpu.VMEM((CHUNK,), jnp.int32),
            pltpu.VMEM((CHUNK,), jnp.int32),
            pltpu.VMEM((CHUNK, ROWW), jnp.float32),
            pltpu.VMEM((ZROWS, ROWW), jnp.float32),
            pltpu.VMEM_SHARED((RB + NTRASH, ROWW), jnp.float32),
            pltpu.SemaphoreType.DMA,
            pltpu.SemaphoreType.DMA,
        ],
        compiler_params=_SC_PARAMS,
    )
    return kfn(msgext, idxp)


def _zpn_body(mem_ref, wzT_ref, s_ref, out_ref):
    zp = jnp.dot(mem_ref[...], wzT_ref[...], preferred_element_type=jnp.float32)
    nrm = jnp.sqrt(jnp.sum(zp * zp, axis=1, keepdims=True))
    out_ref[...] = zp / (nrm + 1e-6) * s_ref[0, 0]


def _alpha_body(feats_ref, delta_ref, zg_ref, wfT_ref, wdp_ref, c0_ref, out_ref):
    f = feats_ref[...]
    fp = jnp.dot(f, wfT_ref[...], preferred_element_type=jnp.float32)
    nrm = jnp.sqrt(jnp.sum(fp * fp, axis=1, keepdims=True))
    fpn = fp / (nrm + 1e-6)
    core = jnp.sum(fpn * zg_ref[...], axis=1, keepdims=True)
    dt = jnp.sum(delta_ref[...] * wdp_ref[...], axis=1, keepdims=True) + c0_ref[0, 0]
    a = jnp.exp(core + dt)
    out_ref[...] = jnp.concatenate(
        [f * a, a, jnp.ones_like(a), jnp.zeros((f.shape[0], ROWW - D - 2), jnp.float32)],
        axis=1)


def _vox_body(mem_ref, we_ref, wg1aT, wg1bT, bg1, wg2T, bg2,
              wihT, whhT, bih, bhh, lng, lnb, wd1T, bd1, wd2T, bd2, wd3T, bd3,
              out_ref):
    m = mem_ref[...]
    we = we_ref[...]
    wsum = we[:, D:D + 1]
    cnt = we[:, D + 1:D + 2]
    msg = we[:, :D] / (wsum + 1e-8)

    h = jax.nn.relu(
        jnp.dot(m, wg1aT[...], preferred_element_type=jnp.float32)
        + jnp.dot(msg, wg1bT[...], preferred_element_type=jnp.float32)
        + bg1[...])
    g = jax.nn.sigmoid(
        jnp.dot(h, wg2T[...], preferred_element_type=jnp.float32) + bg2[0, 0])

    gi = jnp.dot(msg, wihT[...], preferred_element_type=jnp.float32) + bih[...]
    gh = jnp.dot(m, whhT[...], preferred_element_type=jnp.float32) + bhh[...]
    r = jax.nn.sigmoid(gi[:, :D] + gh[:, :D])
    u = jax.nn.sigmoid(gi[:, D:2 * D] + gh[:, D:2 * D])
    n = jnp.tanh(gi[:, 2 * D:] + r * gh[:, 2 * D:])
    hn = (1.0 - u) * n + u * m
    upd = (cnt > 0).astype(jnp.float32)
    zn = m + upd * g * (hn - m)

    mu = jnp.mean(zn, axis=1, keepdims=True)
    var = jnp.mean((zn - mu) ** 2, axis=1, keepdims=True)
    x = (zn - mu) / jnp.sqrt(var + 1e-5) * lng[...] + lnb[...]

    h1 = jax.nn.relu(jnp.dot(x, wd1T[...], preferred_element_type=jnp.float32) + bd1[...])
    h2 = h1 + jax.nn.relu(jnp.dot(h1, wd2T[...], preferred_element_type=jnp.float32) + bd2[...])
    logit = jnp.dot(h2, wd3T[...], preferred_element_type=jnp.float32) + bd3[0, 0]
    out_ref[...] = jax.nn.sigmoid(logit)


def _rep(shape):
    return pl.BlockSpec(shape, lambda i: (0, 0))


def kernel(mem, feats, delta, idx, Wf, Wz, wd, bd, log_temp, Wg1, bg1, Wg2, bg2,
           W_ih, W_hh, b_ih, b_hh, ln_g, ln_b, Wd1, bd1, Wd2, bd2, Wd3, bd3):
    f32 = jnp.float32
    s = jnp.exp(log_temp) / TAU

    wzT = Wz.T
    wfT = Wf.T
    wdp = jnp.pad(s * wd, (0, 1)).reshape(1, 4)
    c0 = (s * bd).reshape(1, 1)
    wg1aT = Wg1[:, :D].T
    wg1bT = Wg1[:, D:].T
    bg1r = bg1.reshape(1, -1)
    wg2T = Wg2.T
    bg2r = bg2.reshape(1, 1)
    wihT = W_ih.T
    whhT = W_hh.T
    bihr = b_ih.reshape(1, -1)
    bhhr = b_hh.reshape(1, -1)
    lngr = ln_g.reshape(1, -1)
    lnbr = ln_b.reshape(1, -1)
    wd1T = Wd1.T
    bd1r = bd1.reshape(1, -1)
    wd2T = Wd2.T
    bd2r = bd2.reshape(1, -1)
    wd3T = Wd3.T
    bd3r = bd3.reshape(1, 1)
    sr = s.reshape(1, 1)

    idxp = jnp.concatenate([idx.astype(jnp.int32),
                            jnp.zeros((NPAD - N,), jnp.int32)])
    deltap = jnp.pad(delta, ((0, 0), (0, 1)))

    BM = 1024
    zpn = pl.pallas_call(
        _zpn_body,
        grid=(M // BM,),
        in_specs=[pl.BlockSpec((BM, D), lambda i: (i, 0)),
                  _rep((D, P)), _rep((1, 1))],
        out_specs=pl.BlockSpec((BM, P), lambda i: (i, 0)),
        out_shape=jax.ShapeDtypeStruct((M, P), f32),
    )(mem, wzT, sr)

    zg = _zg_gather(zpn, idxp)

    BN = 4000
    msgext = pl.pallas_call(
        _alpha_body,
        grid=(N // BN,),
        in_specs=[pl.BlockSpec((BN, D), lambda i: (i, 0)),
                  pl.BlockSpec((BN, 4), lambda i: (i, 0)),
                  pl.BlockSpec((BN, P), lambda i: (i, 0)),
                  _rep((D, P)), _rep((1, 4)), _rep((1, 1))],
        out_specs=pl.BlockSpec((BN, ROWW), lambda i: (i, 0)),
        out_shape=jax.ShapeDtypeStruct((N, ROWW), f32),
    )(feats, deltap, zg, wfT, wdp, c0)

    wsext = _seg_sum(msgext, idxp)

    GH = Wg1.shape[0]
    DH = Wd1.shape[0]
    out2 = pl.pallas_call(
        _vox_body,
        grid=(M // BM,),
        in_specs=[pl.BlockSpec((BM, D), lambda i: (i, 0)),
                  pl.BlockSpec((BM, ROWW), lambda i: (i, 0)),
                  _rep((D, GH)), _rep((D, GH)), _rep((1, GH)),
                  _rep((GH, 1)), _rep((1, 1)),
                  _rep((D, 3 * D)), _rep((D, 3 * D)),
                  _rep((1, 3 * D)), _rep((1, 3 * D)),
                  _rep((1, D)), _rep((1, D)),
                  _rep((D, DH)), _rep((1, DH)),
                  _rep((DH, DH)), _rep((1, DH)),
                  _rep((DH, 1)), _rep((1, 1))],
        out_specs=pl.BlockSpec((BM, 1), lambda i: (i, 0)),
        out_shape=jax.ShapeDtypeStruct((M, 1), f32),
    )(mem, wsext, wg1aT, wg1bT, bg1r, wg2T, bg2r,
      wihT, whhT, bihr, bhhr, lngr, lnbr, wd1T, bd1r, wd2T, bd2r, wd3T, bd3r)

    return out2[:, 0]
